# baseline (speedup 1.0000x reference)
import math

import jax
import jax.numpy as jnp
from jax import lax
from jax.experimental import pallas as pl
from jax.experimental.pallas import tpu as pltpu
from jax.experimental.pallas import tpu_sc as plsc
import numpy as np

_VOCAB = 1000000
_D = 64
_B = 1024
_S = 200
_SCALE = math.sqrt(_D)

_NC = 2
_NS = 16
_NW = _NC * _NS
_SEQ_PER_W = _B // _NW
_HALF = _S // 2


def _make_pe_np(d_model=_D, max_len=_S):
    position = np.arange(0, max_len, dtype=np.float32)[:, None]
    div_term = np.exp(
        np.arange(0, d_model, 2, dtype=np.float32) * -(math.log(10000.0) / d_model))
    pe = np.zeros((max_len, d_model), dtype=np.float32)
    pe[:, 0::2] = np.sin(position * div_term)
    pe[:, 1::2] = np.cos(position * div_term)
    return pe


_PE_NP = _make_pe_np()


_VC = 2048


def _tc_widen(tt):
    def body(tt_ref, out_ref):
        eye = (lax.broadcasted_iota(jnp.int32, (_D, _D), 0)
               == lax.broadcasted_iota(jnp.int32, (_D, _D), 1)).astype(jnp.float32)
        out_ref[:, 0:_D] = lax.dot_general(
            tt_ref[...], eye, (((0,), (0,)), ((), ())),
            preferred_element_type=jnp.float32)

    grid = (_VOCAB + _VC - 1) // _VC
    return pl.pallas_call(
        body,
        grid=(grid,),
        in_specs=[pl.BlockSpec((_D, _VC), lambda i: (0, i))],
        out_specs=pl.BlockSpec((_VC, 2 * _D), lambda i: (i, 0)),
        out_shape=jax.ShapeDtypeStruct((_VOCAB, 2 * _D), jnp.float32),
    )(tt)


def _sc_embed(x2, pe, table):
    mesh = plsc.VectorSubcoreMesh(core_axis_name="c", subcore_axis_name="s")

    @pl.kernel(
        mesh=mesh,
        compiler_params=pltpu.CompilerParams(use_tc_tiling_on_sc=True),
        out_type=jax.ShapeDtypeStruct((_B * _S, _D), jnp.float32),
        scratch_types=[
            pltpu.VMEM((2 * _SEQ_PER_W, _HALF), jnp.int32),
            pltpu.VMEM((_S, _D), jnp.float32),
            pltpu.VMEM((_S, 2 * _D), jnp.float32),
            pltpu.VMEM((_S, 2 * _D), jnp.float32),
            pltpu.VMEM((_S, _D), jnp.float32),
            pltpu.SemaphoreType.DMA,
            pltpu.SemaphoreType.DMA,
        ],
    )
    def body(x_hbm, pe_hbm, table_hbm, out_hbm, idx_v, pe_v, buf0, buf1,
             res, sem0, sem1):
        tview = table_hbm
        wid = lax.axis_index("s") * _NC + lax.axis_index("c")
        seq0 = wid * _SEQ_PER_W

        pltpu.sync_copy(x_hbm.at[pl.ds(wid * 2 * _SEQ_PER_W, 2 * _SEQ_PER_W)],
                        idx_v)
        pltpu.sync_copy(pe_hbm, pe_v)

        bufs = (buf0, buf1)
        sems = (sem0, sem1)

        def start_gather(j, b):
            pltpu.async_copy(tview.at[idx_v.at[2 * j]],
                             bufs[b].at[pl.ds(0, _HALF)], sems[b])
            pltpu.async_copy(tview.at[idx_v.at[2 * j + 1]],
                             bufs[b].at[pl.ds(_HALF, _HALF)], sems[b])

        def wait_gather(j, b):
            pltpu.make_async_copy(tview.at[idx_v.at[2 * j]],
                                  bufs[b].at[pl.ds(0, _HALF)], sems[b]).wait()
            pltpu.make_async_copy(tview.at[idx_v.at[2 * j + 1]],
                                  bufs[b].at[pl.ds(_HALF, _HALF)],
                                  sems[b]).wait()

        start_gather(0, 0)
        start_gather(1, 1)

        def outer(it, carry):
            j0 = it * 2
            for b in range(2):
                j = j0 + b
                wait_gather(j, b)
                buf = bufs[b]

                @plsc.parallel_loop(0, _S, unroll=8)
                def _(i):
                    for k in range(_D // 16):
                        sl = (i, pl.ds(16 * k, 16))
                        res[sl] = buf[sl] * _SCALE + pe_v[sl]

                pltpu.sync_copy(res, out_hbm.at[pl.ds((seq0 + j) * _S, _S)])

                @pl.when(j + 2 < _SEQ_PER_W)
                def _():
                    start_gather(j + 2, b)
            return carry

        lax.fori_loop(0, _SEQ_PER_W // 2, outer, 0)

    return body(x2, pe, table)


def kernel(x, emb_table):
    x2 = x.astype(jnp.int32).reshape(_B * 2, _HALF)
    pe = jnp.asarray(_PE_NP)
    table128 = _tc_widen(emb_table.T)
    flat = _sc_embed(x2, pe, table128)
    return flat.reshape(_B, _S, _D)


# speedup vs baseline: 1.3079x; 1.3079x over previous
import math

import jax
import jax.numpy as jnp
from jax import lax
from jax.experimental import pallas as pl
from jax.experimental.pallas import tpu as pltpu
from jax.experimental.pallas import tpu_sc as plsc
import numpy as np

_VOCAB = 1000000
_D = 64
_B = 1024
_S = 200
_SCALE = math.sqrt(_D)

_NC = 2
_NS = 16
_NW = _NC * _NS
_SEQ_PER_W = _B // _NW
_HALF = _S // 2


def _make_pe_np(d_model=_D, max_len=_S):
    position = np.arange(0, max_len, dtype=np.float32)[:, None]
    div_term = np.exp(
        np.arange(0, d_model, 2, dtype=np.float32) * -(math.log(10000.0) / d_model))
    pe = np.zeros((max_len, d_model), dtype=np.float32)
    pe[:, 0::2] = np.sin(position * div_term)
    pe[:, 1::2] = np.cos(position * div_term)
    return pe


_PE_NP = _make_pe_np()


_VC = 4096


def _tc_widen(tt):
    def body(tt_ref, out_ref):
        out_ref[:, 0:_D] = tt_ref[...].T

    grid = (_VOCAB + _VC - 1) // _VC
    return pl.pallas_call(
        body,
        grid=(grid,),
        in_specs=[pl.BlockSpec((_D, _VC), lambda i: (0, i))],
        out_specs=pl.BlockSpec((_VC, 2 * _D), lambda i: (i, 0)),
        out_shape=jax.ShapeDtypeStruct((_VOCAB, 2 * _D), jnp.float32),
    )(tt)


def _sc_embed(x2, pe, table):
    mesh = plsc.VectorSubcoreMesh(core_axis_name="c", subcore_axis_name="s")

    @pl.kernel(
        mesh=mesh,
        compiler_params=pltpu.CompilerParams(use_tc_tiling_on_sc=True),
        out_type=jax.ShapeDtypeStruct((_B * _S, _D), jnp.float32),
        scratch_types=[
            pltpu.VMEM((2 * _SEQ_PER_W, _HALF), jnp.int32),
            pltpu.VMEM((_S, _D), jnp.float32),
            pltpu.VMEM((_S, 2 * _D), jnp.float32),
            pltpu.VMEM((_S, 2 * _D), jnp.float32),
            pltpu.VMEM((_S, _D), jnp.float32),
            pltpu.SemaphoreType.DMA,
            pltpu.SemaphoreType.DMA,
        ],
    )
    def body(x_hbm, pe_hbm, table_hbm, out_hbm, idx_v, pe_v, buf0, buf1,
             res, sem0, sem1):
        tview = table_hbm
        wid = lax.axis_index("s") * _NC + lax.axis_index("c")
        seq0 = wid * _SEQ_PER_W

        pltpu.sync_copy(x_hbm.at[pl.ds(wid * 2 * _SEQ_PER_W, 2 * _SEQ_PER_W)],
                        idx_v)
        pltpu.sync_copy(pe_hbm, pe_v)

        bufs = (buf0, buf1)
        sems = (sem0, sem1)

        def start_gather(j, b):
            pltpu.async_copy(tview.at[idx_v.at[2 * j]],
                             bufs[b].at[pl.ds(0, _HALF)], sems[b])
            pltpu.async_copy(tview.at[idx_v.at[2 * j + 1]],
                             bufs[b].at[pl.ds(_HALF, _HALF)], sems[b])

        def wait_gather(j, b):
            pltpu.make_async_copy(tview.at[idx_v.at[2 * j]],
                                  bufs[b].at[pl.ds(0, _HALF)], sems[b]).wait()
            pltpu.make_async_copy(tview.at[idx_v.at[2 * j + 1]],
                                  bufs[b].at[pl.ds(_HALF, _HALF)],
                                  sems[b]).wait()

        start_gather(0, 0)
        start_gather(1, 1)

        def outer(it, carry):
            j0 = it * 2
            for b in range(2):
                j = j0 + b
                wait_gather(j, b)
                buf = bufs[b]

                @plsc.parallel_loop(0, _S, unroll=8)
                def _(i):
                    for k in range(_D // 16):
                        sl = (i, pl.ds(16 * k, 16))
                        res[sl] = buf[sl] * _SCALE + pe_v[sl]

                pltpu.sync_copy(res, out_hbm.at[pl.ds((seq0 + j) * _S, _S)])

                @pl.when(j + 2 < _SEQ_PER_W)
                def _():
                    start_gather(j + 2, b)
            return carry

        lax.fori_loop(0, _SEQ_PER_W // 2, outer, 0)

    return body(x2, pe, table)


def kernel(x, emb_table):
    x2 = x.astype(jnp.int32).reshape(_B * 2, _HALF)
    pe = jnp.asarray(_PE_NP)
    table128 = _tc_widen(emb_table.T)
    flat = _sc_embed(x2, pe, table128)
    return flat.reshape(_B, _S, _D)


# speedup vs baseline: 1.5394x; 1.1770x over previous
import math

import jax
import jax.numpy as jnp
from jax import lax
from jax.experimental import pallas as pl
from jax.experimental.pallas import tpu as pltpu
from jax.experimental.pallas import tpu_sc as plsc
import numpy as np

_VOCAB = 1000000
_D = 64
_B = 1024
_S = 200
_SCALE = math.sqrt(_D)

_NC = 2
_NS = 16
_NW = _NC * _NS
_SEQ_PER_W = _B // _NW
_HALF = _S // 2


def _make_pe_np(d_model=_D, max_len=_S):
    position = np.arange(0, max_len, dtype=np.float32)[:, None]
    div_term = np.exp(
        np.arange(0, d_model, 2, dtype=np.float32) * -(math.log(10000.0) / d_model))
    pe = np.zeros((max_len, d_model), dtype=np.float32)
    pe[:, 0::2] = np.sin(position * div_term)
    pe[:, 1::2] = np.cos(position * div_term)
    return pe


_PE_NP = _make_pe_np()


_VC = 8192


def _tc_widen(tt):
    def body(tt_ref, out_ref):
        out_ref[:, 0:_D] = tt_ref[...].T

    grid = (_VOCAB + _VC - 1) // _VC
    return pl.pallas_call(
        body,
        grid=(grid,),
        in_specs=[pl.BlockSpec((_D, _VC), lambda i: (0, i))],
        out_specs=pl.BlockSpec((_VC, 2 * _D), lambda i: (i, 0)),
        out_shape=jax.ShapeDtypeStruct((_VOCAB, 2 * _D), jnp.float32),
    )(tt)


def _sc_embed(x2, pe, table):
    mesh = plsc.VectorSubcoreMesh(core_axis_name="c", subcore_axis_name="s")

    @pl.kernel(
        mesh=mesh,
        compiler_params=pltpu.CompilerParams(use_tc_tiling_on_sc=True),
        out_type=jax.ShapeDtypeStruct((_B * _S, _D), jnp.float32),
        scratch_types=[
            pltpu.VMEM((2 * _SEQ_PER_W, _HALF), jnp.int32),
            pltpu.VMEM((_S, _D), jnp.float32),
            pltpu.VMEM((_S, 2 * _D), jnp.float32),
            pltpu.VMEM((_S, 2 * _D), jnp.float32),
            pltpu.VMEM((_S, _D), jnp.float32),
            pltpu.SemaphoreType.DMA,
            pltpu.SemaphoreType.DMA,
        ],
    )
    def body(x_hbm, pe_hbm, table_hbm, out_hbm, idx_v, pe_v, buf0, buf1,
             res, sem0, sem1):
        tview = table_hbm
        wid = lax.axis_index("s") * _NC + lax.axis_index("c")
        seq0 = wid * _SEQ_PER_W

        pltpu.sync_copy(x_hbm.at[pl.ds(wid * 2 * _SEQ_PER_W, 2 * _SEQ_PER_W)],
                        idx_v)
        pltpu.sync_copy(pe_hbm, pe_v)

        bufs = (buf0, buf1)
        sems = (sem0, sem1)

        def start_gather(j, b):
            pltpu.async_copy(tview.at[idx_v.at[2 * j]],
                             bufs[b].at[pl.ds(0, _HALF)], sems[b])
            pltpu.async_copy(tview.at[idx_v.at[2 * j + 1]],
                             bufs[b].at[pl.ds(_HALF, _HALF)], sems[b])

        def wait_gather(j, b):
            pltpu.make_async_copy(tview.at[idx_v.at[2 * j]],
                                  bufs[b].at[pl.ds(0, _HALF)], sems[b]).wait()
            pltpu.make_async_copy(tview.at[idx_v.at[2 * j + 1]],
                                  bufs[b].at[pl.ds(_HALF, _HALF)],
                                  sems[b]).wait()

        start_gather(0, 0)
        start_gather(1, 1)

        def outer(it, carry):
            j0 = it * 2
            for b in range(2):
                j = j0 + b
                wait_gather(j, b)
                buf = bufs[b]

                @plsc.parallel_loop(0, _S, unroll=8)
                def _(i):
                    for k in range(_D // 16):
                        sl = (i, pl.ds(16 * k, 16))
                        res[sl] = buf[sl] * _SCALE + pe_v[sl]

                pltpu.sync_copy(res, out_hbm.at[pl.ds((seq0 + j) * _S, _S)])

                @pl.when(j + 2 < _SEQ_PER_W)
                def _():
                    start_gather(j + 2, b)
            return carry

        lax.fori_loop(0, _SEQ_PER_W // 2, outer, 0)

    return body(x2, pe, table)


def kernel(x, emb_table):
    x2 = x.astype(jnp.int32).reshape(_B * 2, _HALF)
    pe = jnp.asarray(_PE_NP)
    table128 = _tc_widen(emb_table.T)
    flat = _sc_embed(x2, pe, table128)
    return flat.reshape(_B, _S, _D)


# speedup vs baseline: 1.6069x; 1.0438x over previous
import math

import jax
import jax.numpy as jnp
from jax import lax
from jax.experimental import pallas as pl
from jax.experimental.pallas import tpu as pltpu
from jax.experimental.pallas import tpu_sc as plsc
import numpy as np

_VOCAB = 1000000
_D = 64
_B = 1024
_S = 200
_SCALE = math.sqrt(_D)

_NC = 2
_NS = 16
_NW = _NC * _NS
_SEQ_PER_W = _B // _NW
_HALF = _S // 2


def _make_pe_np(d_model=_D, max_len=_S):
    position = np.arange(0, max_len, dtype=np.float32)[:, None]
    div_term = np.exp(
        np.arange(0, d_model, 2, dtype=np.float32) * -(math.log(10000.0) / d_model))
    pe = np.zeros((max_len, d_model), dtype=np.float32)
    pe[:, 0::2] = np.sin(position * div_term)
    pe[:, 1::2] = np.cos(position * div_term)
    return pe


_PE_NP = _make_pe_np()


_VC = 16384


def _tc_widen(tt):
    def body(tt_ref, out_ref):
        out_ref[:, 0:_D] = tt_ref[...].T

    grid = (_VOCAB + _VC - 1) // _VC
    return pl.pallas_call(
        body,
        grid=(grid,),
        in_specs=[pl.BlockSpec((_D, _VC), lambda i: (0, i))],
        out_specs=pl.BlockSpec((_VC, 2 * _D), lambda i: (i, 0)),
        out_shape=jax.ShapeDtypeStruct((_VOCAB, 2 * _D), jnp.float32),
    )(tt)


def _sc_embed(x2, pe, table):
    mesh = plsc.VectorSubcoreMesh(core_axis_name="c", subcore_axis_name="s")

    @pl.kernel(
        mesh=mesh,
        compiler_params=pltpu.CompilerParams(use_tc_tiling_on_sc=True),
        out_type=jax.ShapeDtypeStruct((_B * _S, _D), jnp.float32),
        scratch_types=[
            pltpu.VMEM((2 * _SEQ_PER_W, _HALF), jnp.int32),
            pltpu.VMEM((_S, _D), jnp.float32),
            pltpu.VMEM((_S, 2 * _D), jnp.float32),
            pltpu.VMEM((_S, 2 * _D), jnp.float32),
            pltpu.VMEM((_S, _D), jnp.float32),
            pltpu.SemaphoreType.DMA,
            pltpu.SemaphoreType.DMA,
        ],
    )
    def body(x_hbm, pe_hbm, table_hbm, out_hbm, idx_v, pe_v, buf0, buf1,
             res, sem0, sem1):
        tview = table_hbm
        wid = lax.axis_index("s") * _NC + lax.axis_index("c")
        seq0 = wid * _SEQ_PER_W

        pltpu.sync_copy(x_hbm.at[pl.ds(wid * 2 * _SEQ_PER_W, 2 * _SEQ_PER_W)],
                        idx_v)
        pltpu.sync_copy(pe_hbm, pe_v)

        bufs = (buf0, buf1)
        sems = (sem0, sem1)

        def start_gather(j, b):
            pltpu.async_copy(tview.at[idx_v.at[2 * j]],
                             bufs[b].at[pl.ds(0, _HALF)], sems[b])
            pltpu.async_copy(tview.at[idx_v.at[2 * j + 1]],
                             bufs[b].at[pl.ds(_HALF, _HALF)], sems[b])

        def wait_gather(j, b):
            pltpu.make_async_copy(tview.at[idx_v.at[2 * j]],
                                  bufs[b].at[pl.ds(0, _HALF)], sems[b]).wait()
            pltpu.make_async_copy(tview.at[idx_v.at[2 * j + 1]],
                                  bufs[b].at[pl.ds(_HALF, _HALF)],
                                  sems[b]).wait()

        start_gather(0, 0)
        start_gather(1, 1)

        def outer(it, carry):
            j0 = it * 2
            for b in range(2):
                j = j0 + b
                wait_gather(j, b)
                buf = bufs[b]

                @plsc.parallel_loop(0, _S, unroll=8)
                def _(i):
                    for k in range(_D // 16):
                        sl = (i, pl.ds(16 * k, 16))
                        res[sl] = buf[sl] * _SCALE + pe_v[sl]

                pltpu.sync_copy(res, out_hbm.at[pl.ds((seq0 + j) * _S, _S)])

                @pl.when(j + 2 < _SEQ_PER_W)
                def _():
                    start_gather(j + 2, b)
            return carry

        lax.fori_loop(0, _SEQ_PER_W // 2, outer, 0)

    return body(x2, pe, table)


def kernel(x, emb_table):
    x2 = x.astype(jnp.int32).reshape(_B * 2, _HALF)
    pe = jnp.asarray(_PE_NP)
    table128 = _tc_widen(emb_table.T)
    flat = _sc_embed(x2, pe, table128)
    return flat.reshape(_B, _S, _D)
